# Initial kernel scaffold; baseline (speedup 1.0000x reference)
import jax
import jax.numpy as jnp
from jax.experimental import pallas as pl


def kernel(tables, x):
    raise NotImplementedError("write your pallas kernel here")


# speedup vs baseline: 1.1502x; 1.1502x over previous
import functools

import jax
import jax.numpy as jnp
from jax import lax
from jax.experimental import pallas as pl
from jax.experimental.pallas import tpu as pltpu
from jax.experimental.pallas import tpu_sc as plsc

_NUM_CORES = 2
_NUM_SUBCORES = 16
_NUM_WORKERS = _NUM_CORES * _NUM_SUBCORES

_IDX_W = 128
_GATHERS_PER_CHUNK = 13
_CHUNK = _IDX_W * _GATHERS_PER_CHUNK


@functools.cache
def _build(n_rows, total, dim):
    rows_per_worker = total // _NUM_WORKERS
    idx_rows = rows_per_worker // _IDX_W
    n_chunks = rows_per_worker // _CHUNK
    assert rows_per_worker % _CHUNK == 0

    mesh = plsc.VectorSubcoreMesh(
        core_axis_name="c", subcore_axis_name="s",
        num_cores=_NUM_CORES, num_subcores=_NUM_SUBCORES)

    @functools.partial(
        pl.kernel,
        out_type=jax.ShapeDtypeStruct((total, dim), jnp.float32),
        mesh=mesh,
        scratch_types=[
            pltpu.VMEM((idx_rows, _IDX_W), jnp.int32),
            pltpu.VMEM((_CHUNK, dim), jnp.float32),
            pltpu.SemaphoreType.DMA,
        ],
        compiler_params=pltpu.CompilerParams(use_tc_tiling_on_sc=False),
    )
    def gather_kernel(tbl_hbm, idx_hbm, out_hbm, idx_v, rows_v, sem):
        cid = lax.axis_index("c")
        sid = lax.axis_index("s")
        wid = sid * _NUM_CORES + cid
        pltpu.sync_copy(idx_hbm.at[wid], idx_v)
        base = wid * rows_per_worker

        def chunk_body(ci, carry):
            copies = []
            for j in range(_GATHERS_PER_CHUNK):
                cp = pltpu.async_copy(
                    tbl_hbm.at[idx_v.at[ci * _GATHERS_PER_CHUNK + j]],
                    rows_v.at[pl.ds(j * _IDX_W, _IDX_W)],
                    sem)
                copies.append(cp)
            for cp in copies:
                cp.wait()
            pltpu.sync_copy(rows_v, out_hbm.at[pl.ds(base + ci * _CHUNK, _CHUNK)])
            return carry

        lax.fori_loop(0, n_chunks, chunk_body, 0)

    return gather_kernel


def kernel(tables, x):
    n_fields, vocab, dim = tables.shape
    batch, _ = x.shape
    total = batch * n_fields

    tbl = tables.reshape(n_fields * vocab, dim)
    offsets = (jnp.arange(n_fields, dtype=jnp.int32) * vocab)[None, :]
    flat_idx = (x + offsets).reshape(
        _NUM_WORKERS, total // (_NUM_WORKERS * _IDX_W), _IDX_W)

    out = _build(n_fields * vocab, total, dim)(tbl, flat_idx)
    return out.reshape(batch, n_fields, dim)
